# baseline (speedup 1.0000x reference)
import jax
import jax.numpy as jnp
from jax import lax
from jax.experimental import pallas as pl
from jax.experimental.pallas import tpu as pltpu
from jax.experimental.pallas import tpu_sc as plsc

N_FEAT = 10000
H = 64
G3 = 3 * H
NW = 32
CHUNK = 320
NPAD = NW * CHUNK
QW = G3 * H // 4
REC = 512
L = 16
HPAD = 128


def _sigmoid(x):
    return 1.0 / (1.0 + jnp.exp(-x))


def _tanh(x):
    return 1.0 - 2.0 / (jnp.exp(2.0 * x) + 1.0)


def _sc_body(mask_hbm, rec_hbm, u4_hbm,
             hout_hbm, hsum_hbm, cnt_hbm,
             mask_v, idx_v, qbufa, qbufb, recrows, hgbuf, ubuf, hsum_v, cnt_v,
             sema, semb):
    wid = lax.axis_index("s") * 2 + lax.axis_index("c")
    base = wid * CHUNK
    iota = lax.iota(jnp.int32, L)
    zeros = jnp.zeros((L,), jnp.float32)

    pltpu.sync_copy(mask_hbm.at[pl.ds(base, CHUNK)], mask_v)
    nvalid = jnp.minimum(N_FEAT - base, CHUNK)

    def _zi(i, _):
        idx_v[pl.ds(i * L, L)] = iota * 0
        return 0
    lax.fori_loop(0, (CHUNK + 2 * L) // L, _zi, 0)

    def _zh(j, _):
        hsum_v[pl.ds(j * L, L)] = zeros
        return 0
    lax.fori_loop(0, H, _zh, 0)

    def _cp(g, cnt):
        m = mask_v[pl.ds(g * L, L)]
        mb = m != 0
        pos = cnt + plsc.cumsum(m) - 1
        plsc.store_scatter(idx_v, [pos], iota + g * L, mask=mb)
        return cnt + jnp.sum(m)
    cnt = lax.fori_loop(0, CHUNK // L, _cp, jnp.int32(0))

    def _zcol(j, _):
        plsc.store_scatter(hgbuf, [iota, iota * 0 + j], zeros)
        return 0
    lax.fori_loop(0, HPAD, _zcol, 0)

    def _zout(i, _):
        pltpu.sync_copy(hgbuf, hout_hbm.at[pl.ds(base + i * L, L)])
        return 0
    lax.fori_loop(0, nvalid // L, _zout, 0)

    nch = (cnt + (L - 1)) // L
    one = iota * 0 + 1

    def _ig_of(c):
        il = plsc.load_gather(idx_v, [iota + c * L])
        return il + base

    qb = [qbufa, qbufb]
    sm = [sema, semb]

    @pl.when(nch > 0)
    def _prime():
        ig0 = _ig_of(0)
        pltpu.async_copy(u4_hbm.at[ig0 * 4 + 0], qbufa, sema)
        pltpu.async_copy(u4_hbm.at[ig0 * 4 + 1], qbufb, semb)

    def _chunk(c, carry):
        valid = (iota + c * L) < cnt
        validf = jnp.where(valid, 1.0, 0.0)
        ig = _ig_of(c)
        iout = jnp.where(valid, ig, N_FEAT + wid)

        pltpu.sync_copy(rec_hbm.at[ig], recrows)
        xv = plsc.load_gather(recrows, [iota, iota * 0 + 448])

        for q in range(4):
            qbuf = qb[q % 2]
            pltpu.make_async_copy(u4_hbm.at[ig * 4 + q], qbuf, sm[q % 2]).wait()
            for half in range(2):
                hk = [plsc.load_gather(
                          recrows, [iota, iota * 0 + (384 + half * 32 + k)])
                      for k in range(32)]

                def _g(gl, _, q=q, half=half, hk=hk, qbuf=qbuf):
                    if half == 0:
                        acc = zeros
                    else:
                        acc = ubuf[pl.ds((q * 48 + gl) * L, L)]
                    off = iota * 0 + (gl * H + half * 32)
                    for k in range(32):
                        uv = plsc.load_gather(qbuf, [iota, off])
                        off = off + one
                        acc = acc + uv * hk[k]
                    ubuf[pl.ds((q * 48 + gl) * L, L)] = acc
                    return 0
                lax.fori_loop(0, 48, _g, 0)
            if q < 2:
                pltpu.async_copy(u4_hbm.at[ig * 4 + (q + 2)], qbuf, sm[q % 2])
            else:
                @pl.when(c + 1 < nch)
                def _pf(q=q, qbuf=qbuf):
                    ign = _ig_of(c + 1)
                    pltpu.async_copy(u4_hbm.at[ign * 4 + (q - 2)], qbuf,
                                     sm[q % 2])

        def _j(j, idxs):
            i_xwz, i_xwr, i_xwh, i_bz, i_br, i_bh, i_hp = idxs
            uz = ubuf[pl.ds(j * L, L)]
            ur = ubuf[pl.ds((H + j) * L, L)]
            uh = ubuf[pl.ds((2 * H + j) * L, L)]
            xwz = plsc.load_gather(recrows, [iota, i_xwz])
            xwr = plsc.load_gather(recrows, [iota, i_xwr])
            xwh = plsc.load_gather(recrows, [iota, i_xwh])
            bz = plsc.load_gather(recrows, [iota, i_bz])
            br = plsc.load_gather(recrows, [iota, i_br])
            bh = plsc.load_gather(recrows, [iota, i_bh])
            hp = plsc.load_gather(recrows, [iota, i_hp])
            z = _sigmoid(xwz * xv + bz + uz)
            r = _sigmoid(xwr * xv + br + ur)
            ht_ = _tanh(xwh * xv + bh + r * uh)
            hg = z * hp + (1.0 - z) * ht_
            plsc.store_scatter(hgbuf, [iota, i_xwz], hg)
            hs = hsum_v[pl.ds(j * L, L)]
            hsum_v[pl.ds(j * L, L)] = hs + hg * validf
            return tuple(v + one for v in idxs)
        lax.fori_loop(0, H, _j,
                      tuple(iota * 0 + c0 for c0 in
                            (0, H, 2 * H, G3, G3 + H, G3 + 2 * H, 2 * G3)))

        pltpu.sync_copy(hgbuf, hout_hbm.at[iout])
        return carry
    lax.fori_loop(0, nch, _chunk, 0)

    pltpu.sync_copy(hsum_v, hsum_hbm.at[wid])

    def _zc(i, _):
        cnt_v[pl.ds(i * L, L)] = iota * 0 + cnt
        return 0
    lax.fori_loop(0, HPAD // L, _zc, 0)
    pltpu.sync_copy(cnt_v, cnt_hbm.at[wid])


def _head_body(hs_ref, cnt_ref, w1_ref, b1_ref, w2_ref, b2_ref, pred_ref):
    hs = hs_ref[...]
    hsum = jnp.sum(hs.reshape(NW, H, L), axis=(0, 2)).reshape(H, 1)
    count = jnp.sum(cnt_ref[...].astype(jnp.float32)) / HPAD
    mean = hsum / count
    h1 = jax.nn.relu(jnp.dot(w1_ref[...], mean,
                             preferred_element_type=jnp.float32) + b1_ref[...])
    logits = jnp.dot(w2_ref[...], h1,
                     preferred_element_type=jnp.float32) + b2_ref[...]
    e = jnp.exp(logits - jnp.max(logits, axis=0, keepdims=True))
    p = e / jnp.sum(e, axis=0, keepdims=True)
    pred_ref[...] = jnp.pad(p, ((0, 6), (0, 0)))


def kernel(tim, X, X_hap, mask, Ht, gru_xT_weights, gru_xT_bias, gru_U_weights,
           mlp_W1, mlp_b1, mlp_W2, mlp_b2):
    mask_i = jnp.pad(mask.astype(jnp.int32), (0, NPAD - N_FEAT))
    xw = gru_xT_weights.reshape(N_FEAT, G3)
    rec = jnp.concatenate(
        [xw, gru_xT_bias, Ht, X[:, None],
         jnp.zeros((N_FEAT, REC - 2 * G3 - H - 1), jnp.float32)], axis=1)
    u4 = gru_U_weights.reshape(N_FEAT * 4, QW)

    mesh = plsc.VectorSubcoreMesh(core_axis_name="c", subcore_axis_name="s")
    sc = pl.kernel(
        _sc_body,
        out_type=(
            jax.ShapeDtypeStruct((N_FEAT + NW, HPAD), jnp.float32),
            jax.ShapeDtypeStruct((NW, H * L), jnp.float32),
            jax.ShapeDtypeStruct((NW, HPAD), jnp.int32),
        ),
        mesh=mesh,
        compiler_params=pltpu.CompilerParams(needs_layout_passes=False),
        scratch_types=[
            pltpu.VMEM((CHUNK,), jnp.int32),
            pltpu.VMEM((CHUNK + 2 * L,), jnp.int32),
            pltpu.VMEM((L, QW), jnp.float32),
            pltpu.VMEM((L, QW), jnp.float32),
            pltpu.VMEM((L, REC), jnp.float32),
            pltpu.VMEM((L, HPAD), jnp.float32),
            pltpu.VMEM((G3 * L,), jnp.float32),
            pltpu.VMEM((H * L,), jnp.float32),
            pltpu.VMEM((HPAD,), jnp.int32),
            pltpu.SemaphoreType.DMA,
            pltpu.SemaphoreType.DMA,
        ],
    )
    h_pad, hsum_p, cnt_p = sc(mask_i, rec, u4)

    pred_pad = pl.pallas_call(
        _head_body,
        out_shape=jax.ShapeDtypeStruct((8, 1), jnp.float32),
        in_specs=[
            pl.BlockSpec((NW, H * L), lambda: (0, 0)),
            pl.BlockSpec((NW, HPAD), lambda: (0, 0)),
            pl.BlockSpec((H, H), lambda: (0, 0)),
            pl.BlockSpec((H, 1), lambda: (0, 0)),
            pl.BlockSpec((2, H), lambda: (0, 0)),
            pl.BlockSpec((2, 1), lambda: (0, 0)),
        ],
        out_specs=pl.BlockSpec((8, 1), lambda: (0, 0)),
    )(hsum_p, cnt_p, mlp_W1, mlp_b1.reshape(H, 1), mlp_W2, mlp_b2.reshape(2, 1))

    h_curr = jnp.where(jnp.any(mask), h_pad[:N_FEAT, :H], Ht)
    return pred_pad[:2, 0], h_curr
